# Initial kernel scaffold; baseline (speedup 1.0000x reference)
import functools

import jax
import jax.numpy as jnp
from jax import lax
from jax.experimental import pallas as pl
from jax.experimental.pallas import tpu as pltpu
from jax.experimental.pallas import tpu_sc as plsc

L = 16
B = 256


def _build(N, D, C, NC, NS):
  NW = NC * NS
  mesh = plsc.VectorSubcoreMesh(core_axis_name="c", subcore_axis_name="s")

  @functools.partial(
      pl.kernel,
      mesh=mesh,
      out_type=jax.ShapeDtypeStruct((N, D), jnp.float32),
      scratch_types=[
          pltpu.VMEM((B,), jnp.int32),
          pltpu.VMEM((B,), jnp.int32),
          pltpu.VMEM((B + L,), jnp.int32),
          pltpu.VMEM((B + L,), jnp.int32),
          pltpu.VMEM((B + L,), jnp.int32),
          pltpu.VMEM((B + 2 * L, D), jnp.float32),
          pltpu.SemaphoreType.DMA,
          pltpu.SemaphoreType.DMA,
      ],
  )
  def k(x_hbm, y_hbm, m_hbm, emb_hbm, out_hbm,
        y_v, m_v, mg_v, ms_v, us_v, rows_v, gsem, ssem):
    wid = lax.axis_index("s") * NC + lax.axis_index("c")

    def do_chunk(kk, _):
      c = wid + kk * NW
      base = c * B
      pltpu.sync_copy(y_hbm.at[pl.ds(base, B)], y_v)
      pltpu.sync_copy(m_hbm.at[pl.ds(base, B)], m_v)

      moff = jnp.int32(0)
      uoff = jnp.int32(0)
      for g in range(B // L):
        mv = m_v[pl.ds(g * L, L)]
        yv = y_v[pl.ds(g * L, L)]
        gvec = base + g * L + lax.iota(jnp.int32, L)
        valid = gvec < N
        mb = (mv != 0) & valid
        ub = (mv == 0) & valid
        mcum = plsc.cumsum(mb.astype(jnp.int32))
        ucum = plsc.cumsum(ub.astype(jnp.int32))
        plsc.store_scatter(mg_v, [moff + mcum - 1], yv, mb)
        plsc.store_scatter(ms_v, [moff + mcum - 1], gvec, mb)
        plsc.store_scatter(us_v, [uoff + ucum - 1], gvec, ub)
        moff = moff + jnp.sum(mb.astype(jnp.int32))
        uoff = uoff + jnp.sum(ub.astype(jnp.int32))

      mg_v[pl.ds(moff, L)] = jnp.full((L,), mg_v[0], jnp.int32)
      ms_v[pl.ds(moff, L)] = jnp.full((L,), ms_v[0], jnp.int32)
      us_v[pl.ds(uoff, L)] = jnp.full((L,), us_v[0], jnp.int32)

      n_m = (moff + L - 1) // L
      n_u = (uoff + L - 1) // L
      mpad = n_m * L

      def fire_mg(j, _):
        iv = mg_v[pl.ds(j * L, L)]
        pltpu.make_async_copy(
            emb_hbm.at[iv], rows_v.at[pl.ds(j * L, L)], gsem).start()
        return 0
      lax.fori_loop(0, n_m, fire_mg, 0)

      def fire_ug(j, _):
        iv = us_v[pl.ds(j * L, L)]
        pltpu.make_async_copy(
            x_hbm.at[iv], rows_v.at[pl.ds(mpad + j * L, L)], gsem).start()
        return 0
      lax.fori_loop(0, n_u, fire_ug, 0)

      def drain_mg(j, _):
        iv = mg_v[pl.ds(j * L, L)]
        pltpu.make_async_copy(
            emb_hbm.at[iv], rows_v.at[pl.ds(j * L, L)], gsem).wait()
        return 0
      lax.fori_loop(0, n_m, drain_mg, 0)

      def drain_ug(j, _):
        iv = us_v[pl.ds(j * L, L)]
        pltpu.make_async_copy(
            x_hbm.at[iv], rows_v.at[pl.ds(mpad + j * L, L)], gsem).wait()
        return 0
      lax.fori_loop(0, n_u, drain_ug, 0)

      def fire_sm(j, _):
        sv = ms_v[pl.ds(j * L, L)]
        pltpu.make_async_copy(
            rows_v.at[pl.ds(j * L, L)], out_hbm.at[sv], ssem).start()
        return 0
      lax.fori_loop(0, n_m, fire_sm, 0)

      def fire_su(j, _):
        sv = us_v[pl.ds(j * L, L)]
        pltpu.make_async_copy(
            rows_v.at[pl.ds(mpad + j * L, L)], out_hbm.at[sv], ssem).start()
        return 0
      lax.fori_loop(0, n_u, fire_su, 0)

      def drain_sm(j, _):
        sv = ms_v[pl.ds(j * L, L)]
        pltpu.make_async_copy(
            rows_v.at[pl.ds(j * L, L)], out_hbm.at[sv], ssem).wait()
        return 0
      lax.fori_loop(0, n_m, drain_sm, 0)

      def drain_su(j, _):
        sv = us_v[pl.ds(j * L, L)]
        pltpu.make_async_copy(
            rows_v.at[pl.ds(mpad + j * L, L)], out_hbm.at[sv], ssem).wait()
        return 0
      lax.fori_loop(0, n_u, drain_su, 0)
      return 0

    nct = (C + NW - 1 - wid) // NW
    lax.fori_loop(0, nct, do_chunk, 0)

  return k


def kernel(x, y, mask, emb_weight):
  N, D = x.shape
  C = (N + B - 1) // B
  info = plsc.get_sparse_core_info()
  NC, NS = info.num_cores, info.num_subcores
  pad = C * B - N
  y32 = jnp.pad(y.astype(jnp.int32), (0, pad))
  m32 = jnp.pad(mask.astype(jnp.int32), (0, pad))
  k = _build(N, D, C, NC, NS)
  return k(x, y32, m32, emb_weight)


# speedup vs baseline: 2.6854x; 2.6854x over previous
import functools

import jax
import jax.numpy as jnp
from jax import lax
from jax.experimental import pallas as pl
from jax.experimental.pallas import tpu as pltpu
from jax.experimental.pallas import tpu_sc as plsc

L = 16
B = 256


def _build(N, D, C, NC, NS):
  NW = NC * NS
  mesh = plsc.VectorSubcoreMesh(core_axis_name="c", subcore_axis_name="s")

  @functools.partial(
      pl.kernel,
      mesh=mesh,
      compiler_params=pltpu.CompilerParams(needs_layout_passes=False),
      out_type=jax.ShapeDtypeStruct((N, D), jnp.float32),
      scratch_types=[
          pltpu.VMEM((B,), jnp.int32),
          pltpu.VMEM((B,), jnp.int32),
          pltpu.VMEM((B + L,), jnp.int32),
          pltpu.VMEM((B + L,), jnp.int32),
          pltpu.VMEM((B + L,), jnp.int32),
          pltpu.VMEM((B + 2 * L, D), jnp.float32),
          pltpu.SemaphoreType.DMA,
          pltpu.SemaphoreType.DMA,
      ],
  )
  def k(x_hbm, y_hbm, m_hbm, emb_hbm, out_hbm,
        y_v, m_v, mg_v, ms_v, us_v, rows_v, gsem, ssem):
    wid = lax.axis_index("s") * NC + lax.axis_index("c")

    def do_chunk(kk, _):
      c = wid + kk * NW
      base = c * B
      pltpu.sync_copy(y_hbm.at[pl.ds(base, B)], y_v)
      pltpu.sync_copy(m_hbm.at[pl.ds(base, B)], m_v)

      moff = jnp.int32(0)
      uoff = jnp.int32(0)
      for g in range(B // L):
        mv = m_v[pl.ds(g * L, L)]
        yv = y_v[pl.ds(g * L, L)]
        gvec = base + g * L + lax.iota(jnp.int32, L)
        valid = gvec < N
        mb = (mv != 0) & valid
        ub = (mv == 0) & valid
        mcum = plsc.cumsum(mb.astype(jnp.int32))
        ucum = plsc.cumsum(ub.astype(jnp.int32))
        plsc.store_scatter(mg_v, [moff + mcum - 1], yv, mask=mb)
        plsc.store_scatter(ms_v, [moff + mcum - 1], gvec, mask=mb)
        plsc.store_scatter(us_v, [uoff + ucum - 1], gvec, mask=ub)
        moff = moff + jnp.sum(mb.astype(jnp.int32))
        uoff = uoff + jnp.sum(ub.astype(jnp.int32))

      pad_iota = lax.iota(jnp.int32, L)
      plsc.store_scatter(mg_v, [moff + pad_iota],
                         jnp.full((L,), mg_v[pl.ds(0, L)][0], jnp.int32))
      plsc.store_scatter(ms_v, [moff + pad_iota],
                         jnp.full((L,), ms_v[pl.ds(0, L)][0], jnp.int32))
      plsc.store_scatter(us_v, [uoff + pad_iota],
                         jnp.full((L,), us_v[pl.ds(0, L)][0], jnp.int32))

      n_m = (moff + L - 1) // L
      n_u = (uoff + L - 1) // L

      def fire_mg(j, _):
        iv = mg_v[pl.ds(j * L, L)]
        pltpu.make_async_copy(
            emb_hbm.at[iv], rows_v.at[pl.ds(j * L, L)], gsem).start()
        return 0
      lax.fori_loop(0, n_m, fire_mg, 0)

      def fire_ug(j, _):
        iv = us_v[pl.ds(j * L, L)]
        pltpu.make_async_copy(
            x_hbm.at[iv], rows_v.at[pl.ds((n_m + j) * L, L)], gsem).start()
        return 0
      lax.fori_loop(0, n_u, fire_ug, 0)

      def drain_mg(j, _):
        iv = mg_v[pl.ds(j * L, L)]
        pltpu.make_async_copy(
            emb_hbm.at[iv], rows_v.at[pl.ds(j * L, L)], gsem).wait()
        return 0
      lax.fori_loop(0, n_m, drain_mg, 0)

      def drain_ug(j, _):
        iv = us_v[pl.ds(j * L, L)]
        pltpu.make_async_copy(
            x_hbm.at[iv], rows_v.at[pl.ds((n_m + j) * L, L)], gsem).wait()
        return 0
      lax.fori_loop(0, n_u, drain_ug, 0)

      def fire_sm(j, _):
        sv = ms_v[pl.ds(j * L, L)]
        pltpu.make_async_copy(
            rows_v.at[pl.ds(j * L, L)], out_hbm.at[sv], ssem).start()
        return 0
      lax.fori_loop(0, n_m, fire_sm, 0)

      def fire_su(j, _):
        sv = us_v[pl.ds(j * L, L)]
        pltpu.make_async_copy(
            rows_v.at[pl.ds((n_m + j) * L, L)], out_hbm.at[sv], ssem).start()
        return 0
      lax.fori_loop(0, n_u, fire_su, 0)

      def drain_sm(j, _):
        sv = ms_v[pl.ds(j * L, L)]
        pltpu.make_async_copy(
            rows_v.at[pl.ds(j * L, L)], out_hbm.at[sv], ssem).wait()
        return 0
      lax.fori_loop(0, n_m, drain_sm, 0)

      def drain_su(j, _):
        sv = us_v[pl.ds(j * L, L)]
        pltpu.make_async_copy(
            rows_v.at[pl.ds((n_m + j) * L, L)], out_hbm.at[sv], ssem).wait()
        return 0
      lax.fori_loop(0, n_u, drain_su, 0)
      return 0

    nct = (C + NW - 1 - wid) // NW
    lax.fori_loop(0, nct, do_chunk, 0)

  return k


def kernel(x, y, mask, emb_weight):
  N, D = x.shape
  C = (N + B - 1) // B
  info = plsc.get_sparse_core_info()
  NC, NS = info.num_cores, info.num_subcores
  pad = C * B - N
  y32 = jnp.pad(y.astype(jnp.int32), (0, pad))
  m32 = jnp.pad(mask.astype(jnp.int32), (0, pad))
  k = _build(N, D, C, NC, NS)
  return k(x, y32, m32, emb_weight)


# speedup vs baseline: 3.1255x; 1.1639x over previous
import functools

import jax
import jax.numpy as jnp
from jax import lax
from jax.experimental import pallas as pl
from jax.experimental.pallas import tpu as pltpu
from jax.experimental.pallas import tpu_sc as plsc

L = 16
B = 256


def _build(N, D, C, NC, NS):
  NW = NC * NS
  mesh = plsc.VectorSubcoreMesh(core_axis_name="c", subcore_axis_name="s")
  idx_t = pltpu.VMEM((B + L,), jnp.int32)
  rows_t = pltpu.VMEM((B + 2 * L, D), jnp.float32)

  @functools.partial(
      pl.kernel,
      mesh=mesh,
      compiler_params=pltpu.CompilerParams(needs_layout_passes=False),
      out_type=jax.ShapeDtypeStruct((N, D), jnp.float32),
      scratch_types=[
          pltpu.VMEM((B,), jnp.int32),
          pltpu.VMEM((B,), jnp.int32),
          idx_t, idx_t, idx_t, rows_t,
          idx_t, idx_t, idx_t, rows_t,
          pltpu.SemaphoreType.DMA, pltpu.SemaphoreType.DMA,
          pltpu.SemaphoreType.DMA, pltpu.SemaphoreType.DMA,
      ],
  )
  def k(x_hbm, y_hbm, m_hbm, emb_hbm, out_hbm,
        y_v, m_v,
        mgA, msA, usA, rowsA,
        mgB, msB, usB, rowsB,
        gsemA, gsemB, ssemA, ssemB):
    wid = lax.axis_index("s") * NC + lax.axis_index("c")
    nct = (C + NW - 1 - wid) // NW

    def compact(kchunk, mg_v, ms_v, us_v):
      c = jnp.minimum(wid + kchunk * NW, jnp.int32(C - 1))
      base = c * B
      pltpu.sync_copy(y_hbm.at[pl.ds(base, B)], y_v)
      pltpu.sync_copy(m_hbm.at[pl.ds(base, B)], m_v)
      moff = jnp.int32(0)
      uoff = jnp.int32(0)
      for g in range(B // L):
        mv = m_v[pl.ds(g * L, L)]
        yv = y_v[pl.ds(g * L, L)]
        gvec = base + g * L + lax.iota(jnp.int32, L)
        valid = gvec < N
        mb = (mv != 0) & valid
        ub = (mv == 0) & valid
        mi = jnp.where(mb, 1, 0)
        ui = jnp.where(ub, 1, 0)
        mcum = plsc.cumsum(mi)
        ucum = plsc.cumsum(ui)
        plsc.store_scatter(mg_v, [moff + mcum - 1], yv, mask=mb)
        plsc.store_scatter(ms_v, [moff + mcum - 1], gvec, mask=mb)
        plsc.store_scatter(us_v, [uoff + ucum - 1], gvec, mask=ub)
        moff = moff + jnp.sum(mi)
        uoff = uoff + jnp.sum(ui)
      pad_iota = lax.iota(jnp.int32, L)
      plsc.store_scatter(mg_v, [moff + pad_iota],
                         jnp.full((L,), mg_v[pl.ds(0, L)][0], jnp.int32))
      plsc.store_scatter(ms_v, [moff + pad_iota],
                         jnp.full((L,), ms_v[pl.ds(0, L)][0], jnp.int32))
      plsc.store_scatter(us_v, [uoff + pad_iota],
                         jnp.full((L,), us_v[pl.ds(0, L)][0], jnp.int32))
      live = kchunk < nct
      n_m = jnp.where(live, (moff + L - 1) // L, 0)
      n_u = jnp.where(live, (uoff + L - 1) // L, 0)
      return n_m, n_u

    def fire_gathers(n_m, n_u, mg_v, us_v, rows_v, gsem):
      def fm(j, _):
        iv = mg_v[pl.ds(j * L, L)]
        pltpu.make_async_copy(
            emb_hbm.at[iv], rows_v.at[pl.ds(j * L, L)], gsem).start()
        return 0
      lax.fori_loop(0, n_m, fm, 0)

      def fu(j, _):
        iv = us_v[pl.ds(j * L, L)]
        pltpu.make_async_copy(
            x_hbm.at[iv], rows_v.at[pl.ds((n_m + j) * L, L)], gsem).start()
        return 0
      lax.fori_loop(0, n_u, fu, 0)

    def drain_gathers(n_m, n_u, mg_v, us_v, rows_v, gsem):
      def dm(j, _):
        iv = mg_v[pl.ds(j * L, L)]
        pltpu.make_async_copy(
            emb_hbm.at[iv], rows_v.at[pl.ds(j * L, L)], gsem).wait()
        return 0
      lax.fori_loop(0, n_m, dm, 0)

      def du(j, _):
        iv = us_v[pl.ds(j * L, L)]
        pltpu.make_async_copy(
            x_hbm.at[iv], rows_v.at[pl.ds((n_m + j) * L, L)], gsem).wait()
        return 0
      lax.fori_loop(0, n_u, du, 0)

    def fire_scatters(n_m, n_u, ms_v, us_v, rows_v, ssem):
      def fm(j, _):
        sv = ms_v[pl.ds(j * L, L)]
        pltpu.make_async_copy(
            rows_v.at[pl.ds(j * L, L)], out_hbm.at[sv], ssem).start()
        return 0
      lax.fori_loop(0, n_m, fm, 0)

      def fu(j, _):
        sv = us_v[pl.ds(j * L, L)]
        pltpu.make_async_copy(
            rows_v.at[pl.ds((n_m + j) * L, L)], out_hbm.at[sv], ssem).start()
        return 0
      lax.fori_loop(0, n_u, fu, 0)

    def drain_scatters(n_m, n_u, ms_v, us_v, rows_v, ssem):
      def dm(j, _):
        sv = ms_v[pl.ds(j * L, L)]
        pltpu.make_async_copy(
            rows_v.at[pl.ds(j * L, L)], out_hbm.at[sv], ssem).wait()
        return 0
      lax.fori_loop(0, n_m, dm, 0)

      def du(j, _):
        sv = us_v[pl.ds(j * L, L)]
        pltpu.make_async_copy(
            rows_v.at[pl.ds((n_m + j) * L, L)], out_hbm.at[sv], ssem).wait()
        return 0
      lax.fori_loop(0, n_u, du, 0)

    bufA = (mgA, msA, usA, rowsA, gsemA, ssemA)
    bufB = (mgB, msB, usB, rowsB, gsemB, ssemB)

    def phase(kchunk, cur, nxt, cur_counts, nxt_prev_counts):
      mg_c, ms_c, us_c, rows_c, gsem_c, ssem_c = cur
      mg_n, ms_n, us_n, rows_n, gsem_n, ssem_n = nxt
      drain_scatters(nxt_prev_counts[0], nxt_prev_counts[1],
                     ms_n, us_n, rows_n, ssem_n)
      n_m1, n_u1 = compact(kchunk + 1, mg_n, ms_n, us_n)
      fire_gathers(n_m1, n_u1, mg_n, us_n, rows_n, gsem_n)
      drain_gathers(cur_counts[0], cur_counts[1], mg_c, us_c, rows_c, gsem_c)
      fire_scatters(cur_counts[0], cur_counts[1], ms_c, us_c, rows_c, ssem_c)
      return (n_m1, n_u1)

    nmA, nuA = compact(0, mgA, msA, usA)
    fire_gathers(nmA, nuA, mgA, usA, rowsA, gsemA)

    npairs = (nct + 2 + 1) // 2

    def pair(kp, carry):
      cA, cB = carry
      cB2 = phase(2 * kp, bufA, bufB, cA, cB)
      cA2 = phase(2 * kp + 1, bufB, bufA, cB2, cA)
      return (cA2, cB2)

    zero = (jnp.int32(0), jnp.int32(0))
    (cA, cB) = lax.fori_loop(0, npairs, pair, ((nmA, nuA), zero))
    drain_scatters(cA[0], cA[1], msA, usA, rowsA, ssemA)
    drain_scatters(cB[0], cB[1], msB, usB, rowsB, ssemB)

  return k


def kernel(x, y, mask, emb_weight):
  N, D = x.shape
  C = (N + B - 1) // B
  info = plsc.get_sparse_core_info()
  NC, NS = info.num_cores, info.num_subcores
  pad = C * B - N
  y32 = jnp.pad(y.astype(jnp.int32), (0, pad))
  m32 = jnp.pad(mask.astype(jnp.int32), (0, pad))
  k = _build(N, D, C, NC, NS)
  return k(x, y32, m32, emb_weight)
